# Initial kernel scaffold; baseline (speedup 1.0000x reference)
import jax
import jax.numpy as jnp
from jax.experimental import pallas as pl


def kernel(x_vals, x_cons, edge_index, batch_vals, batch_cons, params):
    raise NotImplementedError("write your pallas kernel here")


# speedup vs baseline: 1.5647x; 1.5647x over previous
import functools

import jax
import jax.numpy as jnp
from jax import lax
from jax.experimental import pallas as pl
from jax.experimental.pallas import tpu as pltpu
from jax.experimental.pallas import tpu_sc as plsc

_HID = 128
_N = 10000
_E = 320000
_G = 64
_NC = 2
_NS = 16
_NW = _NC * _NS
_EPW = _E // _NW
_EPW2 = 10240
_CH = 128
_NCHUNK = _EPW2 // _CH
_IRW = 2 * _NCHUNK
_ACC = 10248
_CCH = 80
_CNCH = _EPW // _CCH
_NPAD = 10240
_RPT = _NPAD // _NS
_ZR = 128
_CW = 16
_NF = _HID // _CW
_BR = 1000


def _relu(x):
    return jnp.maximum(x, 0.0)


def _dot(a, b):
    return lax.dot_general(a, b, (((1,), (0,)), ((), ())),
                           preferred_element_type=jnp.float32)


def _id_rows(idbuf, base):
    lane = jnp.arange(16, dtype=jnp.int32)
    for i in range(_NCHUNK // 16):
        idbuf[pl.ds(i * 16, 16)] = lane + (base + i * 16)


def _sc_agg_body(tab_hbm, idx_hbm, osum, idbuf, idg, ids, rows, zbuf, acc, sem):
    c = lax.axis_index("c")
    s = lax.axis_index("s")
    w = s * _NC + c

    _id_rows(idbuf, w * _IRW)
    pltpu.async_copy(idx_hbm.at[idbuf], idg, sem).wait()
    _id_rows(idbuf, w * _IRW + _NCHUNK)
    pltpu.async_copy(idx_hbm.at[idbuf], ids, sem).wait()

    def _fill(i, carry):
        zbuf[i, :] = jnp.zeros((_CW,), jnp.float32)
        return carry
    lax.fori_loop(0, _RPT, _fill, 0)
    row0 = pl.multiple_of(s * _RPT, 8)

    for f in range(_NF):
        pltpu.sync_copy(zbuf, acc.at[pl.ds(row0, _RPT)])
        plsc.subcore_barrier()

        def _chunk(k, carry):
            pltpu.async_copy(tab_hbm.at[f].at[idg.at[k]], rows, sem).wait()
            pltpu.sync_copy(rows, acc.at[ids.at[k]], add=True)
            return carry
        lax.fori_loop(0, _NCHUNK, _chunk, 0)

        plsc.subcore_barrier()
        pltpu.sync_copy(acc.at[pl.ds(row0, _RPT)],
                        osum.at[c, f, pl.ds(row0, _RPT)])
        plsc.subcore_barrier()


def _sc_segment_sum(table8, comb_idx):
    kern = pl.kernel(
        _sc_agg_body,
        out_type=jax.ShapeDtypeStruct((_NC, _NF, _NPAD, _CW), jnp.float32),
        mesh=plsc.VectorSubcoreMesh(core_axis_name="c", subcore_axis_name="s"),
        compiler_params=pltpu.CompilerParams(use_tc_tiling_on_sc=False),
        scratch_types=[
            pltpu.VMEM((_NCHUNK,), jnp.int32),
            pltpu.VMEM((_NCHUNK, _CH), jnp.int32),
            pltpu.VMEM((_NCHUNK, _CH), jnp.int32),
            pltpu.VMEM((_CH, _CW), jnp.float32),
            pltpu.VMEM((_RPT, _CW), jnp.float32),
            pltpu.VMEM_SHARED((_ACC, _CW), jnp.float32),
            pltpu.SemaphoreType.DMA,
        ],
    )
    return kern(table8, comb_idx)


def _sc_counts_body(idx_hbm, out_hbm, idbuf, idd, idsrc, ones_v, zbuf,
                    acc_d, acc_s, sem):
    c = lax.axis_index("c")
    s = lax.axis_index("s")
    w = s * _NC + c

    _id_rows(idbuf, w * _IRW)
    pltpu.async_copy(idx_hbm.at[idbuf], idd, sem).wait()
    _id_rows(idbuf, w * _IRW + _NCHUNK)
    pltpu.async_copy(idx_hbm.at[idbuf], idsrc, sem).wait()

    def _fill(i, carry):
        ones_v[i, :] = jnp.ones((_CW,), jnp.float32)
        return carry
    lax.fori_loop(0, _CH, _fill, 0)

    def _zero(i, carry):
        zbuf[i, :] = jnp.zeros((_CW,), jnp.float32)
        return carry
    lax.fori_loop(0, _RPT, _zero, 0)

    row0 = pl.multiple_of(s * _RPT, 8)
    pltpu.sync_copy(zbuf, acc_d.at[pl.ds(row0, _RPT)])
    pltpu.sync_copy(zbuf, acc_s.at[pl.ds(row0, _RPT)])
    plsc.subcore_barrier()

    def _step(k, carry):
        pltpu.sync_copy(ones_v, acc_d.at[idd.at[k]], add=True)
        pltpu.sync_copy(ones_v, acc_s.at[idsrc.at[k]], add=True)
        return carry
    lax.fori_loop(0, _NCHUNK, _step, 0)

    plsc.subcore_barrier()
    pltpu.sync_copy(acc_d.at[pl.ds(row0, _RPT)], out_hbm.at[0, c, pl.ds(row0, _RPT)])
    pltpu.sync_copy(acc_s.at[pl.ds(row0, _RPT)], out_hbm.at[1, c, pl.ds(row0, _RPT)])


def _sc_counts(comb_idx):
    kern = pl.kernel(
        _sc_counts_body,
        out_type=jax.ShapeDtypeStruct((2, _NC, _NPAD, _CW), jnp.float32),
        mesh=plsc.VectorSubcoreMesh(core_axis_name="c", subcore_axis_name="s"),
        compiler_params=pltpu.CompilerParams(use_tc_tiling_on_sc=False),
        scratch_types=[
            pltpu.VMEM((_NCHUNK,), jnp.int32),
            pltpu.VMEM((_NCHUNK, _CH), jnp.int32),
            pltpu.VMEM((_NCHUNK, _CH), jnp.int32),
            pltpu.VMEM((_CH, _CW), jnp.float32),
            pltpu.VMEM((_RPT, _CW), jnp.float32),
            pltpu.VMEM_SHARED((_ACC, _CW), jnp.float32),
            pltpu.VMEM_SHARED((_ACC, _CW), jnp.float32),
            pltpu.SemaphoreType.DMA,
        ],
    )
    return kern(comb_idx)


def _enc_msg_body(x_ref, w1_ref, b1_ref, w2_ref, b2_ref, wm_ref, bm_ref,
                  h_ref, m_ref):
    h = _relu(_dot(x_ref[...], w1_ref[...]) + b1_ref[...])
    h = _relu(_dot(h, w2_ref[...]) + b2_ref[...])
    h_ref[...] = h
    m_ref[...] = _relu(_dot(h, wm_ref[...]) + bm_ref[...])


def _enc_body(x_ref, w1_ref, b1_ref, w2_ref, b2_ref, h_ref):
    h = _relu(_dot(x_ref[...], w1_ref[...]) + b1_ref[...])
    h_ref[...] = _relu(_dot(h, w2_ref[...]) + b2_ref[...])


_bs_x = pl.BlockSpec((_BR, _HID), lambda i: (i, 0))
_bs_w = pl.BlockSpec((_HID, _HID), lambda i: (0, 0))
_bs_b = pl.BlockSpec((1, _HID), lambda i: (0, 0))
_bs_wu = pl.BlockSpec((2 * _HID, _HID), lambda i: (0, 0))
_bs_p = pl.BlockSpec((_NC, _BR, _HID), lambda i: (0, i, 0))
_bs_c = pl.BlockSpec((_NC, _BR, _CW), lambda i: (0, i, 0))
_sds = functools.partial(jax.ShapeDtypeStruct, dtype=jnp.float32)


def _tc_encode_msg(x, enc, pm):
    (w1, b1), (w2, b2) = enc
    wm, bm = pm
    return pl.pallas_call(
        _enc_msg_body,
        grid=(_N // _BR,),
        in_specs=[_bs_x, _bs_w, _bs_b, _bs_w, _bs_b, _bs_w, _bs_b],
        out_specs=[_bs_x, _bs_x],
        out_shape=[_sds((_N, _HID))] * 2,
    )(x, w1, b1.reshape(1, _HID), w2, b2.reshape(1, _HID),
      wm, bm.reshape(1, _HID))


def _tc_encode(x, enc):
    (w1, b1), (w2, b2) = enc
    return pl.pallas_call(
        _enc_body,
        grid=(_N // _BR,),
        in_specs=[_bs_x, _bs_w, _bs_b, _bs_w, _bs_b],
        out_specs=_bs_x,
        out_shape=_sds((_N, _HID)),
    )(x, w1, b1.reshape(1, _HID), w2, b2.reshape(1, _HID))


def _ln_update(h_ref, s_ref, c_ref, wu_ref, bu_ref):
    ssum = s_ref[0] + s_ref[1]
    cnt = c_ref[0, :, 0:1] + c_ref[1, :, 0:1]
    agg = ssum / jnp.maximum(cnt, 1.0)
    wu = wu_ref[...]
    t = _relu(_dot(h_ref[...], wu[:_HID]) + _dot(agg, wu[_HID:]) + bu_ref[...])
    mu = jnp.mean(t, axis=-1, keepdims=True)
    xc = t - mu
    var = jnp.mean(xc * xc, axis=-1, keepdims=True)
    return xc * lax.rsqrt(var + 1e-5)


def _upd_msg_body(h_ref, s_ref, c_ref, wu_ref, bu_ref, wm_ref, bm_ref,
                  h_out, m_out):
    hn = _ln_update(h_ref, s_ref, c_ref, wu_ref, bu_ref)
    h_out[...] = hn
    m_out[...] = _relu(_dot(hn, wm_ref[...]) + bm_ref[...])


def _upd_body(h_ref, s_ref, c_ref, wu_ref, bu_ref, h_out):
    h_out[...] = _ln_update(h_ref, s_ref, c_ref, wu_ref, bu_ref)


def _tc_update_msg(h, p, cnt, upd, pm):
    wu, bu = upd
    wm, bm = pm
    return pl.pallas_call(
        _upd_msg_body,
        grid=(_N // _BR,),
        in_specs=[_bs_x, _bs_p, _bs_c, _bs_wu, _bs_b, _bs_w, _bs_b],
        out_specs=[_bs_x, _bs_x],
        out_shape=[_sds((_N, _HID))] * 2,
    )(h, p, cnt, wu, bu.reshape(1, _HID), wm, bm.reshape(1, _HID))


def _tc_update(h, p, cnt, upd):
    wu, bu = upd
    return pl.pallas_call(
        _upd_body,
        grid=(_N // _BR,),
        in_specs=[_bs_x, _bs_p, _bs_c, _bs_wu, _bs_b],
        out_specs=_bs_x,
        out_shape=_sds((_N, _HID)),
    )(h, p, cnt, wu, bu.reshape(1, _HID))


def _pool_pred_body(hv_ref, hc_ref, bv_ref, bc_ref, w1_ref, b1_ref,
                    w2_ref, b2_ref, out_ref):
    gid = lax.broadcasted_iota(jnp.int32, (_G, _N), 0)

    def _pool(h_ref, b_ref):
        oh = (b_ref[...] == gid).astype(jnp.float32)
        ssum = _dot(oh, h_ref[...])
        cnt = jnp.sum(oh, axis=1, keepdims=True)
        return ssum / jnp.maximum(cnt, 1.0)

    e = jnp.concatenate([_pool(hv_ref, bv_ref), _pool(hc_ref, bc_ref)], axis=1)
    e = _relu(_dot(e, w1_ref[...]) + b1_ref[...])
    out_ref[...] = _dot(e, w2_ref[...]) + b2_ref[...]


def _tc_pool_pred(h_v, h_c, bv, bc, pred):
    (w1, b1), (w2, b2) = pred
    return pl.pallas_call(
        _pool_pred_body,
        out_shape=_sds((_G, _HID)),
    )(h_v, h_c, bv.reshape(1, _N), bc.reshape(1, _N),
      w1, b1.reshape(1, _HID), w2, b2.reshape(1, _HID))


def _to8(m):
    return m.reshape(_N, _NF, _CW).transpose(1, 0, 2)


def _from8(p8):
    return p8.transpose(0, 2, 1, 3).reshape(_NC, _NPAD, _HID)


def kernel(x_vals, x_cons, edge_index, batch_vals, batch_cons, params):
    src_w = edge_index[0].astype(jnp.int32).reshape(_NW, _EPW)
    dst_w = edge_index[1].astype(jnp.int32).reshape(_NW, _EPW)
    padg = jnp.zeros((_NW, _EPW2 - _EPW), jnp.int32)
    pads = jnp.full((_NW, _EPW2 - _EPW), 10240, jnp.int32)
    src_g = jnp.concatenate([src_w, padg], axis=1)
    src_s = jnp.concatenate([src_w, pads], axis=1)
    dst_g = jnp.concatenate([dst_w, padg], axis=1)
    dst_s = jnp.concatenate([dst_w, pads], axis=1)
    comb_sd = jnp.concatenate([src_g, dst_s], axis=1).reshape(_NW * _IRW, _CH)
    comb_ds = jnp.concatenate([dst_g, src_s], axis=1).reshape(_NW * _IRW, _CH)
    comb_cc = jnp.concatenate([dst_s, src_s], axis=1).reshape(_NW * _IRW, _CH)

    cnts = _sc_counts(comb_cc)
    cnt_d = cnts[0]
    cnt_s = cnts[1]

    h_v, m = _tc_encode_msg(x_vals, params['enc_vals'],
                            params['convs'][0]['v2c_msg'])
    h_c = _tc_encode(x_cons, params['enc_cons'])

    for li, layer in enumerate(params['convs']):
        p = _from8(_sc_segment_sum(_to8(m), comb_sd))
        h_c, m = _tc_update_msg(h_c, p, cnt_d, layer['v2c_upd'],
                                layer['c2v_msg'])
        p = _from8(_sc_segment_sum(_to8(m), comb_ds))
        if li + 1 < len(params['convs']):
            h_v, m = _tc_update_msg(h_v, p, cnt_s, layer['c2v_upd'],
                                    params['convs'][li + 1]['v2c_msg'])
        else:
            h_v = _tc_update(h_v, p, cnt_s, layer['c2v_upd'])

    return _tc_pool_pred(h_v, h_c, batch_vals.astype(jnp.int32),
                         batch_cons.astype(jnp.int32), params['pred'])


# speedup vs baseline: 1.9963x; 1.2759x over previous
import functools

import jax
import jax.numpy as jnp
from jax import lax
from jax.experimental import pallas as pl
from jax.experimental.pallas import tpu as pltpu
from jax.experimental.pallas import tpu_sc as plsc

_HID = 128
_N = 10000
_E = 320000
_G = 64
_NC = 2
_NS = 16
_NW = _NC * _NS
_EPW = _E // _NW
_EPW2 = 10240
_CH = 128
_NCHUNK = _EPW2 // _CH
_IRW = 2 * _NCHUNK
_ACC = 10248
_CCH = 80
_CNCH = _EPW // _CCH
_NPAD = 10240
_RPT = _NPAD // _NS
_ZR = 128
_CW = 16
_NF = _HID // _CW
_BR = 1000


def _relu(x):
    return jnp.maximum(x, 0.0)


def _dot(a, b):
    return lax.dot_general(a, b, (((1,), (0,)), ((), ())),
                           preferred_element_type=jnp.float32)


def _id_rows(idbuf, base):
    lane = jnp.arange(16, dtype=jnp.int32)
    for i in range(_NCHUNK // 16):
        idbuf[pl.ds(i * 16, 16)] = lane + (base + i * 16)


def _sc_agg_body(tab_hbm, idx_hbm, osum, idbuf, idg, ids, rows, rows_b,
                 zbuf, acc, sem, sem_b):
    c = lax.axis_index("c")
    s = lax.axis_index("s")
    w = s * _NC + c

    _id_rows(idbuf, w * _IRW)
    pltpu.async_copy(idx_hbm.at[idbuf], idg, sem).wait()
    _id_rows(idbuf, w * _IRW + _NCHUNK)
    pltpu.async_copy(idx_hbm.at[idbuf], ids, sem).wait()

    def _fill(i, carry):
        zbuf[i, :] = jnp.zeros((_CW,), jnp.float32)
        return carry
    lax.fori_loop(0, _RPT, _fill, 0)
    row0 = pl.multiple_of(s * _RPT, 8)

    for f in range(_NF):
        pltpu.sync_copy(zbuf, acc.at[pl.ds(row0, _RPT)])
        plsc.subcore_barrier()

        def _chunk(i, carry):
            k0 = 2 * i
            k1 = 2 * i + 1
            cp0 = pltpu.async_copy(tab_hbm.at[f].at[idg.at[k0]], rows, sem)
            cp1 = pltpu.async_copy(tab_hbm.at[f].at[idg.at[k1]], rows_b, sem_b)
            cp0.wait()
            pltpu.sync_copy(rows, acc.at[ids.at[k0]], add=True)
            cp1.wait()
            pltpu.sync_copy(rows_b, acc.at[ids.at[k1]], add=True)
            return carry
        lax.fori_loop(0, _NCHUNK // 2, _chunk, 0)

        plsc.subcore_barrier()
        pltpu.sync_copy(acc.at[pl.ds(row0, _RPT)],
                        osum.at[c, f, pl.ds(row0, _RPT)])
        plsc.subcore_barrier()


def _sc_segment_sum(table8, comb_idx):
    kern = pl.kernel(
        _sc_agg_body,
        out_type=jax.ShapeDtypeStruct((_NC, _NF, _NPAD, _CW), jnp.float32),
        mesh=plsc.VectorSubcoreMesh(core_axis_name="c", subcore_axis_name="s"),
        compiler_params=pltpu.CompilerParams(use_tc_tiling_on_sc=False),
        scratch_types=[
            pltpu.VMEM((_NCHUNK,), jnp.int32),
            pltpu.VMEM((_NCHUNK, _CH), jnp.int32),
            pltpu.VMEM((_NCHUNK, _CH), jnp.int32),
            pltpu.VMEM((_CH, _CW), jnp.float32),
            pltpu.VMEM((_CH, _CW), jnp.float32),
            pltpu.VMEM((_RPT, _CW), jnp.float32),
            pltpu.VMEM_SHARED((_ACC, _CW), jnp.float32),
            pltpu.SemaphoreType.DMA,
            pltpu.SemaphoreType.DMA,
        ],
    )
    return kern(table8, comb_idx)


def _sc_counts_body(idx_hbm, out_hbm, idbuf, idd, idsrc, ones_v, zbuf,
                    acc_d, acc_s, sem):
    c = lax.axis_index("c")
    s = lax.axis_index("s")
    w = s * _NC + c

    _id_rows(idbuf, w * _IRW)
    pltpu.async_copy(idx_hbm.at[idbuf], idd, sem).wait()
    _id_rows(idbuf, w * _IRW + _NCHUNK)
    pltpu.async_copy(idx_hbm.at[idbuf], idsrc, sem).wait()

    def _fill(i, carry):
        ones_v[i, :] = jnp.ones((_CW,), jnp.float32)
        return carry
    lax.fori_loop(0, _CH, _fill, 0)

    def _zero(i, carry):
        zbuf[i, :] = jnp.zeros((_CW,), jnp.float32)
        return carry
    lax.fori_loop(0, _RPT, _zero, 0)

    row0 = pl.multiple_of(s * _RPT, 8)
    pltpu.sync_copy(zbuf, acc_d.at[pl.ds(row0, _RPT)])
    pltpu.sync_copy(zbuf, acc_s.at[pl.ds(row0, _RPT)])
    plsc.subcore_barrier()

    def _step(k, carry):
        pltpu.sync_copy(ones_v, acc_d.at[idd.at[k]], add=True)
        pltpu.sync_copy(ones_v, acc_s.at[idsrc.at[k]], add=True)
        return carry
    lax.fori_loop(0, _NCHUNK, _step, 0)

    plsc.subcore_barrier()
    pltpu.sync_copy(acc_d.at[pl.ds(row0, _RPT)], out_hbm.at[0, c, pl.ds(row0, _RPT)])
    pltpu.sync_copy(acc_s.at[pl.ds(row0, _RPT)], out_hbm.at[1, c, pl.ds(row0, _RPT)])


def _sc_counts(comb_idx):
    kern = pl.kernel(
        _sc_counts_body,
        out_type=jax.ShapeDtypeStruct((2, _NC, _NPAD, _CW), jnp.float32),
        mesh=plsc.VectorSubcoreMesh(core_axis_name="c", subcore_axis_name="s"),
        compiler_params=pltpu.CompilerParams(use_tc_tiling_on_sc=False),
        scratch_types=[
            pltpu.VMEM((_NCHUNK,), jnp.int32),
            pltpu.VMEM((_NCHUNK, _CH), jnp.int32),
            pltpu.VMEM((_NCHUNK, _CH), jnp.int32),
            pltpu.VMEM((_CH, _CW), jnp.float32),
            pltpu.VMEM((_RPT, _CW), jnp.float32),
            pltpu.VMEM_SHARED((_ACC, _CW), jnp.float32),
            pltpu.VMEM_SHARED((_ACC, _CW), jnp.float32),
            pltpu.SemaphoreType.DMA,
        ],
    )
    return kern(comb_idx)


def _enc_msg_body(x_ref, w1_ref, b1_ref, w2_ref, b2_ref, wm_ref, bm_ref,
                  h_ref, m_ref):
    h = _relu(_dot(x_ref[...], w1_ref[...]) + b1_ref[...])
    h = _relu(_dot(h, w2_ref[...]) + b2_ref[...])
    h_ref[...] = h
    m_ref[...] = _relu(_dot(h, wm_ref[...]) + bm_ref[...])


def _enc_body(x_ref, w1_ref, b1_ref, w2_ref, b2_ref, h_ref):
    h = _relu(_dot(x_ref[...], w1_ref[...]) + b1_ref[...])
    h_ref[...] = _relu(_dot(h, w2_ref[...]) + b2_ref[...])


_bs_x = pl.BlockSpec((_BR, _HID), lambda i: (i, 0))
_bs_w = pl.BlockSpec((_HID, _HID), lambda i: (0, 0))
_bs_b = pl.BlockSpec((1, _HID), lambda i: (0, 0))
_bs_wu = pl.BlockSpec((2 * _HID, _HID), lambda i: (0, 0))
_bs_p = pl.BlockSpec((_NC, _BR, _HID), lambda i: (0, i, 0))
_bs_c = pl.BlockSpec((_NC, _BR, _CW), lambda i: (0, i, 0))
_sds = functools.partial(jax.ShapeDtypeStruct, dtype=jnp.float32)


def _tc_encode_msg(x, enc, pm):
    (w1, b1), (w2, b2) = enc
    wm, bm = pm
    return pl.pallas_call(
        _enc_msg_body,
        grid=(_N // _BR,),
        in_specs=[_bs_x, _bs_w, _bs_b, _bs_w, _bs_b, _bs_w, _bs_b],
        out_specs=[_bs_x, _bs_x],
        out_shape=[_sds((_N, _HID))] * 2,
    )(x, w1, b1.reshape(1, _HID), w2, b2.reshape(1, _HID),
      wm, bm.reshape(1, _HID))


def _tc_encode(x, enc):
    (w1, b1), (w2, b2) = enc
    return pl.pallas_call(
        _enc_body,
        grid=(_N // _BR,),
        in_specs=[_bs_x, _bs_w, _bs_b, _bs_w, _bs_b],
        out_specs=_bs_x,
        out_shape=_sds((_N, _HID)),
    )(x, w1, b1.reshape(1, _HID), w2, b2.reshape(1, _HID))


def _ln_update(h_ref, s_ref, c_ref, wu_ref, bu_ref):
    ssum = s_ref[0] + s_ref[1]
    cnt = c_ref[0, :, 0:1] + c_ref[1, :, 0:1]
    agg = ssum / jnp.maximum(cnt, 1.0)
    wu = wu_ref[...]
    t = _relu(_dot(h_ref[...], wu[:_HID]) + _dot(agg, wu[_HID:]) + bu_ref[...])
    mu = jnp.mean(t, axis=-1, keepdims=True)
    xc = t - mu
    var = jnp.mean(xc * xc, axis=-1, keepdims=True)
    return xc * lax.rsqrt(var + 1e-5)


def _upd_msg_body(h_ref, s_ref, c_ref, wu_ref, bu_ref, wm_ref, bm_ref,
                  h_out, m_out):
    hn = _ln_update(h_ref, s_ref, c_ref, wu_ref, bu_ref)
    h_out[...] = hn
    m_out[...] = _relu(_dot(hn, wm_ref[...]) + bm_ref[...])


def _upd_body(h_ref, s_ref, c_ref, wu_ref, bu_ref, h_out):
    h_out[...] = _ln_update(h_ref, s_ref, c_ref, wu_ref, bu_ref)


def _tc_update_msg(h, p, cnt, upd, pm):
    wu, bu = upd
    wm, bm = pm
    return pl.pallas_call(
        _upd_msg_body,
        grid=(_N // _BR,),
        in_specs=[_bs_x, _bs_p, _bs_c, _bs_wu, _bs_b, _bs_w, _bs_b],
        out_specs=[_bs_x, _bs_x],
        out_shape=[_sds((_N, _HID))] * 2,
    )(h, p, cnt, wu, bu.reshape(1, _HID), wm, bm.reshape(1, _HID))


def _tc_update(h, p, cnt, upd):
    wu, bu = upd
    return pl.pallas_call(
        _upd_body,
        grid=(_N // _BR,),
        in_specs=[_bs_x, _bs_p, _bs_c, _bs_wu, _bs_b],
        out_specs=_bs_x,
        out_shape=_sds((_N, _HID)),
    )(h, p, cnt, wu, bu.reshape(1, _HID))


def _pool_pred_body(hv_ref, hc_ref, bv_ref, bc_ref, w1_ref, b1_ref,
                    w2_ref, b2_ref, out_ref):
    gid = lax.broadcasted_iota(jnp.int32, (_G, _N), 0)

    def _pool(h_ref, b_ref):
        oh = (b_ref[...] == gid).astype(jnp.float32)
        ssum = _dot(oh, h_ref[...])
        cnt = jnp.sum(oh, axis=1, keepdims=True)
        return ssum / jnp.maximum(cnt, 1.0)

    e = jnp.concatenate([_pool(hv_ref, bv_ref), _pool(hc_ref, bc_ref)], axis=1)
    e = _relu(_dot(e, w1_ref[...]) + b1_ref[...])
    out_ref[...] = _dot(e, w2_ref[...]) + b2_ref[...]


def _tc_pool_pred(h_v, h_c, bv, bc, pred):
    (w1, b1), (w2, b2) = pred
    return pl.pallas_call(
        _pool_pred_body,
        out_shape=_sds((_G, _HID)),
    )(h_v, h_c, bv.reshape(1, _N), bc.reshape(1, _N),
      w1, b1.reshape(1, _HID), w2, b2.reshape(1, _HID))


def _to8(m):
    return m.reshape(_N, _NF, _CW).transpose(1, 0, 2)


def _from8(p8):
    return p8.transpose(0, 2, 1, 3).reshape(_NC, _NPAD, _HID)


def kernel(x_vals, x_cons, edge_index, batch_vals, batch_cons, params):
    src_w = edge_index[0].astype(jnp.int32).reshape(_NW, _EPW)
    dst_w = edge_index[1].astype(jnp.int32).reshape(_NW, _EPW)
    padg = jnp.zeros((_NW, _EPW2 - _EPW), jnp.int32)
    pads = jnp.full((_NW, _EPW2 - _EPW), 10240, jnp.int32)
    src_g = jnp.concatenate([src_w, padg], axis=1)
    src_s = jnp.concatenate([src_w, pads], axis=1)
    dst_g = jnp.concatenate([dst_w, padg], axis=1)
    dst_s = jnp.concatenate([dst_w, pads], axis=1)
    comb_sd = jnp.concatenate([src_g, dst_s], axis=1).reshape(_NW * _IRW, _CH)
    comb_ds = jnp.concatenate([dst_g, src_s], axis=1).reshape(_NW * _IRW, _CH)
    comb_cc = jnp.concatenate([dst_s, src_s], axis=1).reshape(_NW * _IRW, _CH)

    cnts = _sc_counts(comb_cc)
    cnt_d = cnts[0]
    cnt_s = cnts[1]

    h_v, m = _tc_encode_msg(x_vals, params['enc_vals'],
                            params['convs'][0]['v2c_msg'])
    h_c = _tc_encode(x_cons, params['enc_cons'])

    for li, layer in enumerate(params['convs']):
        p = _from8(_sc_segment_sum(_to8(m), comb_sd))
        h_c, m = _tc_update_msg(h_c, p, cnt_d, layer['v2c_upd'],
                                layer['c2v_msg'])
        p = _from8(_sc_segment_sum(_to8(m), comb_ds))
        if li + 1 < len(params['convs']):
            h_v, m = _tc_update_msg(h_v, p, cnt_s, layer['c2v_upd'],
                                    params['convs'][li + 1]['v2c_msg'])
        else:
            h_v = _tc_update(h_v, p, cnt_s, layer['c2v_upd'])

    return _tc_pool_pred(h_v, h_c, batch_vals.astype(jnp.int32),
                         batch_cons.astype(jnp.int32), params['pred'])


# speedup vs baseline: 2.2613x; 1.1327x over previous
import functools

import jax
import jax.numpy as jnp
from jax import lax
from jax.experimental import pallas as pl
from jax.experimental.pallas import tpu as pltpu
from jax.experimental.pallas import tpu_sc as plsc

_HID = 128
_N = 10000
_E = 320000
_G = 64
_NC = 2
_NS = 16
_NW = _NC * _NS
_EPW = _E // _NW
_EPW2 = 10240
_CH = 128
_NCHUNK = _EPW2 // _CH
_IRW = 2 * _NCHUNK
_ACC = 10248
_CCH = 80
_CNCH = _EPW // _CCH
_NPAD = 10240
_RPT = _NPAD // _NS
_ZR = 128
_CW = 16
_NF = _HID // _CW
_BR = 1000


def _relu(x):
    return jnp.maximum(x, 0.0)


def _dot(a, b):
    return lax.dot_general(a, b, (((1,), (0,)), ((), ())),
                           preferred_element_type=jnp.float32)


def _id_rows(idbuf, base):
    lane = jnp.arange(16, dtype=jnp.int32)
    for i in range(_NCHUNK // 16):
        idbuf[pl.ds(i * 16, 16)] = lane + (base + i * 16)


def _sc_agg_body(tab_hbm, idx_hbm, osum, idbuf, idg, ids, r0, r1, r2, r3,
                 zbuf, acc, s0, s1, s2, s3):
    rbufs = (r0, r1, r2, r3)
    sems = (s0, s1, s2, s3)
    c = lax.axis_index("c")
    s = lax.axis_index("s")
    w = s * _NC + c

    _id_rows(idbuf, w * _IRW)
    pltpu.async_copy(idx_hbm.at[idbuf], idg, s0).wait()
    _id_rows(idbuf, w * _IRW + _NCHUNK)
    pltpu.async_copy(idx_hbm.at[idbuf], ids, s0).wait()

    def _fill(i, carry):
        zbuf[i, :] = jnp.zeros((_CW,), jnp.float32)
        return carry
    lax.fori_loop(0, _RPT, _fill, 0)
    row0 = pl.multiple_of(s * _RPT, 8)

    for f in range(_NF):
        pltpu.sync_copy(zbuf, acc.at[pl.ds(row0, _RPT)])
        plsc.subcore_barrier()

        def _chunk(i, carry):
            cps = []
            for j in range(4):
                cps.append(pltpu.async_copy(
                    tab_hbm.at[f].at[idg.at[4 * i + j]], rbufs[j], sems[j]))
            for j in range(4):
                cps[j].wait()
                pltpu.sync_copy(rbufs[j], acc.at[ids.at[4 * i + j]], add=True)
            return carry
        lax.fori_loop(0, _NCHUNK // 4, _chunk, 0)

        plsc.subcore_barrier()
        pltpu.sync_copy(acc.at[pl.ds(row0, _RPT)],
                        osum.at[c, f, pl.ds(row0, _RPT)])
        plsc.subcore_barrier()


def _sc_segment_sum(table8, comb_idx):
    kern = pl.kernel(
        _sc_agg_body,
        out_type=jax.ShapeDtypeStruct((_NC, _NF, _NPAD, _CW), jnp.float32),
        mesh=plsc.VectorSubcoreMesh(core_axis_name="c", subcore_axis_name="s"),
        compiler_params=pltpu.CompilerParams(use_tc_tiling_on_sc=False),
        scratch_types=[
            pltpu.VMEM((_NCHUNK,), jnp.int32),
            pltpu.VMEM((_NCHUNK, _CH), jnp.int32),
            pltpu.VMEM((_NCHUNK, _CH), jnp.int32),
            pltpu.VMEM((_CH, _CW), jnp.float32),
            pltpu.VMEM((_CH, _CW), jnp.float32),
            pltpu.VMEM((_CH, _CW), jnp.float32),
            pltpu.VMEM((_CH, _CW), jnp.float32),
            pltpu.VMEM((_RPT, _CW), jnp.float32),
            pltpu.VMEM_SHARED((_ACC, _CW), jnp.float32),
            pltpu.SemaphoreType.DMA,
            pltpu.SemaphoreType.DMA,
            pltpu.SemaphoreType.DMA,
            pltpu.SemaphoreType.DMA,
        ],
    )
    return kern(table8, comb_idx)


def _sc_counts_body(idx_hbm, out_hbm, idbuf, idd, idsrc, ones_v, zbuf,
                    acc_d, acc_s, sem):
    c = lax.axis_index("c")
    s = lax.axis_index("s")
    w = s * _NC + c

    _id_rows(idbuf, w * _IRW)
    pltpu.async_copy(idx_hbm.at[idbuf], idd, sem).wait()
    _id_rows(idbuf, w * _IRW + _NCHUNK)
    pltpu.async_copy(idx_hbm.at[idbuf], idsrc, sem).wait()

    def _fill(i, carry):
        ones_v[i, :] = jnp.ones((_CW,), jnp.float32)
        return carry
    lax.fori_loop(0, _CH, _fill, 0)

    def _zero(i, carry):
        zbuf[i, :] = jnp.zeros((_CW,), jnp.float32)
        return carry
    lax.fori_loop(0, _RPT, _zero, 0)

    row0 = pl.multiple_of(s * _RPT, 8)
    pltpu.sync_copy(zbuf, acc_d.at[pl.ds(row0, _RPT)])
    pltpu.sync_copy(zbuf, acc_s.at[pl.ds(row0, _RPT)])
    plsc.subcore_barrier()

    def _step(k, carry):
        pltpu.sync_copy(ones_v, acc_d.at[idd.at[k]], add=True)
        pltpu.sync_copy(ones_v, acc_s.at[idsrc.at[k]], add=True)
        return carry
    lax.fori_loop(0, _NCHUNK, _step, 0)

    plsc.subcore_barrier()
    pltpu.sync_copy(acc_d.at[pl.ds(row0, _RPT)], out_hbm.at[0, c, pl.ds(row0, _RPT)])
    pltpu.sync_copy(acc_s.at[pl.ds(row0, _RPT)], out_hbm.at[1, c, pl.ds(row0, _RPT)])


def _sc_counts(comb_idx):
    kern = pl.kernel(
        _sc_counts_body,
        out_type=jax.ShapeDtypeStruct((2, _NC, _NPAD, _CW), jnp.float32),
        mesh=plsc.VectorSubcoreMesh(core_axis_name="c", subcore_axis_name="s"),
        compiler_params=pltpu.CompilerParams(use_tc_tiling_on_sc=False),
        scratch_types=[
            pltpu.VMEM((_NCHUNK,), jnp.int32),
            pltpu.VMEM((_NCHUNK, _CH), jnp.int32),
            pltpu.VMEM((_NCHUNK, _CH), jnp.int32),
            pltpu.VMEM((_CH, _CW), jnp.float32),
            pltpu.VMEM((_RPT, _CW), jnp.float32),
            pltpu.VMEM_SHARED((_ACC, _CW), jnp.float32),
            pltpu.VMEM_SHARED((_ACC, _CW), jnp.float32),
            pltpu.SemaphoreType.DMA,
        ],
    )
    return kern(comb_idx)


def _enc_msg_body(x_ref, w1_ref, b1_ref, w2_ref, b2_ref, wm_ref, bm_ref,
                  h_ref, m_ref):
    h = _relu(_dot(x_ref[...], w1_ref[...]) + b1_ref[...])
    h = _relu(_dot(h, w2_ref[...]) + b2_ref[...])
    h_ref[...] = h
    m_ref[...] = _relu(_dot(h, wm_ref[...]) + bm_ref[...])


def _enc_body(x_ref, w1_ref, b1_ref, w2_ref, b2_ref, h_ref):
    h = _relu(_dot(x_ref[...], w1_ref[...]) + b1_ref[...])
    h_ref[...] = _relu(_dot(h, w2_ref[...]) + b2_ref[...])


_bs_x = pl.BlockSpec((_BR, _HID), lambda i: (i, 0))
_bs_w = pl.BlockSpec((_HID, _HID), lambda i: (0, 0))
_bs_b = pl.BlockSpec((1, _HID), lambda i: (0, 0))
_bs_wu = pl.BlockSpec((2 * _HID, _HID), lambda i: (0, 0))
_bs_p = pl.BlockSpec((_NC, _BR, _HID), lambda i: (0, i, 0))
_bs_c = pl.BlockSpec((_NC, _BR, _CW), lambda i: (0, i, 0))
_sds = functools.partial(jax.ShapeDtypeStruct, dtype=jnp.float32)


def _tc_encode_msg(x, enc, pm):
    (w1, b1), (w2, b2) = enc
    wm, bm = pm
    return pl.pallas_call(
        _enc_msg_body,
        grid=(_N // _BR,),
        in_specs=[_bs_x, _bs_w, _bs_b, _bs_w, _bs_b, _bs_w, _bs_b],
        out_specs=[_bs_x, _bs_x],
        out_shape=[_sds((_N, _HID))] * 2,
    )(x, w1, b1.reshape(1, _HID), w2, b2.reshape(1, _HID),
      wm, bm.reshape(1, _HID))


def _tc_encode(x, enc):
    (w1, b1), (w2, b2) = enc
    return pl.pallas_call(
        _enc_body,
        grid=(_N // _BR,),
        in_specs=[_bs_x, _bs_w, _bs_b, _bs_w, _bs_b],
        out_specs=_bs_x,
        out_shape=_sds((_N, _HID)),
    )(x, w1, b1.reshape(1, _HID), w2, b2.reshape(1, _HID))


def _ln_update(h_ref, s_ref, c_ref, wu_ref, bu_ref):
    ssum = s_ref[0] + s_ref[1]
    cnt = c_ref[0, :, 0:1] + c_ref[1, :, 0:1]
    agg = ssum / jnp.maximum(cnt, 1.0)
    wu = wu_ref[...]
    t = _relu(_dot(h_ref[...], wu[:_HID]) + _dot(agg, wu[_HID:]) + bu_ref[...])
    mu = jnp.mean(t, axis=-1, keepdims=True)
    xc = t - mu
    var = jnp.mean(xc * xc, axis=-1, keepdims=True)
    return xc * lax.rsqrt(var + 1e-5)


def _upd_msg_body(h_ref, s_ref, c_ref, wu_ref, bu_ref, wm_ref, bm_ref,
                  h_out, m_out):
    hn = _ln_update(h_ref, s_ref, c_ref, wu_ref, bu_ref)
    h_out[...] = hn
    m_out[...] = _relu(_dot(hn, wm_ref[...]) + bm_ref[...])


def _upd_body(h_ref, s_ref, c_ref, wu_ref, bu_ref, h_out):
    h_out[...] = _ln_update(h_ref, s_ref, c_ref, wu_ref, bu_ref)


def _tc_update_msg(h, p, cnt, upd, pm):
    wu, bu = upd
    wm, bm = pm
    return pl.pallas_call(
        _upd_msg_body,
        grid=(_N // _BR,),
        in_specs=[_bs_x, _bs_p, _bs_c, _bs_wu, _bs_b, _bs_w, _bs_b],
        out_specs=[_bs_x, _bs_x],
        out_shape=[_sds((_N, _HID))] * 2,
    )(h, p, cnt, wu, bu.reshape(1, _HID), wm, bm.reshape(1, _HID))


def _tc_update(h, p, cnt, upd):
    wu, bu = upd
    return pl.pallas_call(
        _upd_body,
        grid=(_N // _BR,),
        in_specs=[_bs_x, _bs_p, _bs_c, _bs_wu, _bs_b],
        out_specs=_bs_x,
        out_shape=_sds((_N, _HID)),
    )(h, p, cnt, wu, bu.reshape(1, _HID))


def _pool_pred_body(hv_ref, hc_ref, bv_ref, bc_ref, w1_ref, b1_ref,
                    w2_ref, b2_ref, out_ref):
    gid = lax.broadcasted_iota(jnp.int32, (_G, _N), 0)

    def _pool(h_ref, b_ref):
        oh = (b_ref[...] == gid).astype(jnp.float32)
        ssum = _dot(oh, h_ref[...])
        cnt = jnp.sum(oh, axis=1, keepdims=True)
        return ssum / jnp.maximum(cnt, 1.0)

    e = jnp.concatenate([_pool(hv_ref, bv_ref), _pool(hc_ref, bc_ref)], axis=1)
    e = _relu(_dot(e, w1_ref[...]) + b1_ref[...])
    out_ref[...] = _dot(e, w2_ref[...]) + b2_ref[...]


def _tc_pool_pred(h_v, h_c, bv, bc, pred):
    (w1, b1), (w2, b2) = pred
    return pl.pallas_call(
        _pool_pred_body,
        out_shape=_sds((_G, _HID)),
    )(h_v, h_c, bv.reshape(1, _N), bc.reshape(1, _N),
      w1, b1.reshape(1, _HID), w2, b2.reshape(1, _HID))


def _to8(m):
    return m.reshape(_N, _NF, _CW).transpose(1, 0, 2)


def _from8(p8):
    return p8.transpose(0, 2, 1, 3).reshape(_NC, _NPAD, _HID)


def kernel(x_vals, x_cons, edge_index, batch_vals, batch_cons, params):
    src_w = edge_index[0].astype(jnp.int32).reshape(_NW, _EPW)
    dst_w = edge_index[1].astype(jnp.int32).reshape(_NW, _EPW)
    padg = jnp.zeros((_NW, _EPW2 - _EPW), jnp.int32)
    pads = jnp.full((_NW, _EPW2 - _EPW), 10240, jnp.int32)
    src_g = jnp.concatenate([src_w, padg], axis=1)
    src_s = jnp.concatenate([src_w, pads], axis=1)
    dst_g = jnp.concatenate([dst_w, padg], axis=1)
    dst_s = jnp.concatenate([dst_w, pads], axis=1)
    comb_sd = jnp.concatenate([src_g, dst_s], axis=1).reshape(_NW * _IRW, _CH)
    comb_ds = jnp.concatenate([dst_g, src_s], axis=1).reshape(_NW * _IRW, _CH)
    comb_cc = jnp.concatenate([dst_s, src_s], axis=1).reshape(_NW * _IRW, _CH)

    cnts = _sc_counts(comb_cc)
    cnt_d = cnts[0]
    cnt_s = cnts[1]

    h_v, m = _tc_encode_msg(x_vals, params['enc_vals'],
                            params['convs'][0]['v2c_msg'])
    h_c = _tc_encode(x_cons, params['enc_cons'])

    for li, layer in enumerate(params['convs']):
        p = _from8(_sc_segment_sum(_to8(m), comb_sd))
        h_c, m = _tc_update_msg(h_c, p, cnt_d, layer['v2c_upd'],
                                layer['c2v_msg'])
        p = _from8(_sc_segment_sum(_to8(m), comb_ds))
        if li + 1 < len(params['convs']):
            h_v, m = _tc_update_msg(h_v, p, cnt_s, layer['c2v_upd'],
                                    params['convs'][li + 1]['v2c_msg'])
        else:
            h_v = _tc_update(h_v, p, cnt_s, layer['c2v_upd'])

    return _tc_pool_pred(h_v, h_c, batch_vals.astype(jnp.int32),
                         batch_cons.astype(jnp.int32), params['pred'])
